# baseline (speedup 1.0000x reference)
import jax
import jax.numpy as jnp
from jax import lax
from jax.experimental import pallas as pl
from jax.experimental.pallas import tpu as pltpu
from jax.experimental.pallas import tpu_sc as plsc

N_NODES = 10000
N_EDGES = 320000

NC = 2
NS = 16
GW = 80
EPAD = 320000
NGRP = EPAD // GW
GRP_PER_TILE = NGRP // NS
NB = 10
NPAD = 10240
ROWS_PER_TILE = NPAD // NS
LROWS = NPAD // 128
LPER = LROWS // NS


def _dense_body(x, W0, b0, gw1, W1, b1, gw2, W2, b2, W3, b3,
                zplan, b1p, c2):
    h0 = jnp.dot(x[...], W0[...], preferred_element_type=jnp.float32) + b0[...]
    t1 = jnp.dot(h0 * gw1[...], W1[...], preferred_element_type=jnp.float32)
    t2 = jnp.dot(t1 * gw2[...], W2[...], preferred_element_type=jnp.float32)
    z = jnp.dot(t2, W3[...], preferred_element_type=jnp.float32)
    zplan[0, 0:N_NODES, :] = z[:, 0:2]
    zplan[1, 0:N_NODES, :] = z[:, 2:4]
    zpad = jnp.zeros((NPAD - N_NODES, 2), jnp.float32)
    zplan[0, N_NODES:NPAD, :] = zpad
    zplan[1, N_NODES:NPAD, :] = zpad
    b1v = b1[...].reshape(1, -1)
    bp = jnp.dot(
        jnp.dot(b1v * gw2[...], W2[...], preferred_element_type=jnp.float32),
        W3[...], preferred_element_type=jnp.float32)
    b1p[...] = bp
    c2[...] = jnp.dot(b2[...].reshape(1, -1), W3[...],
                      preferred_element_type=jnp.float32) + b3[...].reshape(1, -1)


def _agg_round(src_sp, dst_sp, colbuf, rowbuf, vals, gsem, ssem):
    def superbatch(i, _):
        gds = []
        for j in range(NB):
            g = i * NB + j
            gds.append(pltpu.async_copy(
                src_sp.at[colbuf.at[pl.ds(g * GW, GW)]], vals.at[j], gsem))
        for d in gds:
            d.wait()
        sds = []
        for j in range(NB):
            g = i * NB + j
            sds.append(pltpu.async_copy(
                vals.at[j], dst_sp.at[rowbuf.at[pl.ds(g * GW, GW)]],
                ssem, add=True))
        for d in sds:
            d.wait()
        return 0

    lax.fori_loop(0, GRP_PER_TILE // NB, superbatch, 0)


def _iota16():
    return lax.iota(jnp.int32, 16)


def _sc_body(zplan, rowr, colr, zeros2, ident, bconst, out,
             zsp, y1sp, y2sp, colbuf, rowbuf, vals, bcbuf, idbuf,
             gsem, ssem):
    c = lax.axis_index("c")
    s = lax.axis_index("s")
    rbase = s * ROWS_PER_TILE

    pltpu.sync_copy(zplan.at[c, pl.ds(rbase, ROWS_PER_TILE), :],
                    zsp.at[pl.ds(rbase, ROWS_PER_TILE), :])
    ebase = s * (GRP_PER_TILE * GW)
    pltpu.sync_copy(colr.at[pl.ds(ebase, GRP_PER_TILE * GW)], colbuf)
    pltpu.sync_copy(rowr.at[pl.ds(ebase, GRP_PER_TILE * GW)], rowbuf)
    pltpu.sync_copy(bconst.at[c], bcbuf)
    pltpu.sync_copy(ident.at[s], idbuf)
    pltpu.sync_copy(zeros2.at[pl.ds(rbase, ROWS_PER_TILE), :],
                    y1sp.at[pl.ds(rbase, ROWS_PER_TILE), :])
    pltpu.sync_copy(zeros2.at[pl.ds(rbase, ROWS_PER_TILE), :],
                    y2sp.at[pl.ds(rbase, ROWS_PER_TILE), :])
    plsc.subcore_barrier()

    _agg_round(zsp, y1sp, colbuf, rowbuf, vals, gsem, ssem)
    plsc.subcore_barrier()

    for t in range(IDPER):
        pltpu.async_copy(bcbuf, y1sp.at[idbuf.at[t]], ssem, add=True).wait()
    plsc.subcore_barrier()

    _agg_round(y1sp, y2sp, colbuf, rowbuf, vals, gsem, ssem)
    plsc.subcore_barrier()

    pltpu.sync_copy(y2sp.at[pl.ds(rbase, ROWS_PER_TILE), :],
                    out.at[c, pl.ds(rbase, ROWS_PER_TILE), :])


IDCHUNK = 128
IDPER = ROWS_PER_TILE // IDCHUNK


def _softmax_body(y3, c2, out):
    v = jnp.concatenate([y3[0, 0:N_NODES, :], y3[1, 0:N_NODES, :]],
                        axis=-1) + c2[...]
    m = jnp.max(v, axis=-1, keepdims=True)
    e = jnp.exp(v - m)
    out[...] = e / jnp.sum(e, axis=-1, keepdims=True)


def kernel(x, edge_index, W0, b0, gw1, W1, b1, gw2, W2, b2, W3, b3):
    f32 = jnp.float32
    i32 = jnp.int32

    zplan, b1p, c2 = pl.pallas_call(
        _dense_body,
        out_shape=(
            jax.ShapeDtypeStruct((2, NPAD, 2), f32),
            jax.ShapeDtypeStruct((1, 4), f32),
            jax.ShapeDtypeStruct((1, 4), f32),
        ),
    )(x, W0, b0, gw1, W1, b1, gw2, W2, b2, W3, b3)


    row_r = edge_index[0].astype(i32)
    col_r = edge_index[1].astype(i32)
    zeros2 = jnp.zeros((NPAD, 2), f32)
    ident = jnp.arange(NPAD, dtype=i32).reshape(NS, IDPER, IDCHUNK)
    bconst = jnp.broadcast_to(b1p.reshape(2, 1, 2), (2, 128, 2))

    sc_agg = pl.kernel(
        _sc_body,
        out_type=jax.ShapeDtypeStruct((2, NPAD, 2), f32),
        mesh=plsc.VectorSubcoreMesh(
            core_axis_name="c", subcore_axis_name="s"),
        compiler_params=pltpu.CompilerParams(use_tc_tiling_on_sc=False),
        scratch_types=[
            pltpu.VMEM_SHARED((NPAD, 2), f32),
            pltpu.VMEM_SHARED((NPAD, 2), f32),
            pltpu.VMEM_SHARED((NPAD, 2), f32),
            pltpu.VMEM((GRP_PER_TILE * GW,), i32),
            pltpu.VMEM((GRP_PER_TILE * GW,), i32),
            pltpu.VMEM((NB, GW, 2), f32),
            pltpu.VMEM((128, 2), f32),
            pltpu.VMEM((IDPER, IDCHUNK), i32),
            pltpu.SemaphoreType.DMA,
            pltpu.SemaphoreType.DMA,
        ],
    )
    y3 = sc_agg(zplan, row_r, col_r, zeros2, ident, bconst)

    return pl.pallas_call(
        _softmax_body,
        out_shape=jax.ShapeDtypeStruct((N_NODES, 4), f32),
    )(y3, c2)


# speedup vs baseline: 1.4050x; 1.4050x over previous
import functools

import jax
import jax.numpy as jnp
from jax import lax
from jax.experimental import pallas as pl
from jax.experimental.pallas import tpu as pltpu
from jax.experimental.pallas import tpu_sc as plsc

N_NODES = 10000
N_EDGES = 320000

NC = 2
NS = 16
GW = 80
NGRP = N_EDGES // GW
GRP_PER_TILE = NGRP // NS
NB = 10
NPAD = 10240
ROWS_PER_TILE = NPAD // NS
IDCHUNK = 128
IDPER = ROWS_PER_TILE // IDCHUNK


def _dense_body(x, W0, b0, gw1, W1, b1, gw2, W2, b2, W3, b3, z3, b1p, c2):
    h0 = jnp.dot(x[...], W0[...], preferred_element_type=jnp.float32) + b0[...]
    t1 = jnp.dot(h0 * gw1[...], W1[...], preferred_element_type=jnp.float32)
    t2 = jnp.dot(t1 * gw2[...], W2[...], preferred_element_type=jnp.float32)
    z = jnp.dot(t2, W3[...], preferred_element_type=jnp.float32)
    z3[0, 0:N_NODES, :] = z[:, 0:2]
    z3[1, 0:N_NODES, :] = z[:, 2:4]
    b1v = b1[...].reshape(1, -1)
    bp = jnp.dot(
        jnp.dot(b1v * gw2[...], W2[...], preferred_element_type=jnp.float32),
        W3[...], preferred_element_type=jnp.float32)
    b1p[...] = bp
    c2[...] = jnp.dot(b2[...].reshape(1, -1), W3[...],
                      preferred_element_type=jnp.float32) + b3[...].reshape(1, -1)


def _agg_round(src_sp, dst_sp, colbuf, rowbuf, vals, gsem, ssem):
    def superbatch(i, _):
        gds = []
        for j in range(NB):
            g = i * NB + j
            gds.append(pltpu.async_copy(
                src_sp.at[colbuf.at[g]], vals.at[j], gsem))
        for d in gds:
            d.wait()
        sds = []
        for j in range(NB):
            g = i * NB + j
            sds.append(pltpu.async_copy(
                vals.at[j], dst_sp.at[rowbuf.at[g]], ssem, add=True))
        for d in sds:
            d.wait()
        return 0

    lax.fori_loop(0, GRP_PER_TILE // NB, superbatch, 0)


def _sc_body(z3, rowr, colr, zeros2, ident, bconst, out,
             zsp, y1sp, y2sp, colbuf, rowbuf, vals, bcbuf, idbuf, gsem, ssem):
    c = lax.axis_index("c")
    s = lax.axis_index("s")
    rbase = s * ROWS_PER_TILE

    pltpu.sync_copy(z3.at[c, pl.ds(rbase, ROWS_PER_TILE), :],
                    zsp.at[pl.ds(rbase, ROWS_PER_TILE), :])
    pltpu.sync_copy(zeros2.at[pl.ds(rbase, ROWS_PER_TILE), :],
                    y1sp.at[pl.ds(rbase, ROWS_PER_TILE), :])
    pltpu.sync_copy(zeros2.at[pl.ds(rbase, ROWS_PER_TILE), :],
                    y2sp.at[pl.ds(rbase, ROWS_PER_TILE), :])
    pltpu.sync_copy(colr.at[s], colbuf)
    pltpu.sync_copy(rowr.at[s], rowbuf)
    pltpu.sync_copy(bconst.at[c], bcbuf)
    pltpu.sync_copy(ident.at[s], idbuf)
    plsc.subcore_barrier()

    _agg_round(zsp, y1sp, colbuf, rowbuf, vals, gsem, ssem)
    plsc.subcore_barrier()

    for t in range(IDPER):
        pltpu.async_copy(bcbuf, y1sp.at[idbuf.at[t]], ssem, add=True).wait()
    plsc.subcore_barrier()

    _agg_round(y1sp, y2sp, colbuf, rowbuf, vals, gsem, ssem)
    plsc.subcore_barrier()

    pltpu.sync_copy(y2sp.at[pl.ds(rbase, ROWS_PER_TILE), :],
                    out.at[c, pl.ds(rbase, ROWS_PER_TILE), :])


def _softmax_body(y3, c2, out):
    logits = jnp.concatenate([y3[0], y3[1]], axis=-1) + c2[...]
    m = jnp.max(logits, axis=-1, keepdims=True)
    e = jnp.exp(logits - m)
    out[...] = e / jnp.sum(e, axis=-1, keepdims=True)


def kernel(x, edge_index, W0, b0, gw1, W1, b1, gw2, W2, b2, W3, b3):
    f32 = jnp.float32

    z3, b1p, c2 = pl.pallas_call(
        _dense_body,
        out_shape=(
            jax.ShapeDtypeStruct((2, NPAD, 2), f32),
            jax.ShapeDtypeStruct((1, 4), f32),
            jax.ShapeDtypeStruct((1, 4), f32),
        ),
    )(x, W0, b0, gw1, W1, b1, gw2, W2, b2, W3, b3)

    row_r = edge_index[0].astype(jnp.int32).reshape(NS, GRP_PER_TILE, GW)
    col_r = edge_index[1].astype(jnp.int32).reshape(NS, GRP_PER_TILE, GW)
    zeros2 = jnp.zeros((NPAD, 2), f32)
    ident = jnp.arange(NPAD, dtype=jnp.int32).reshape(NS, IDPER, IDCHUNK)
    bconst = jnp.broadcast_to(b1p.reshape(2, 1, 2), (2, 128, 2))

    sc_agg = pl.kernel(
        _sc_body,
        out_type=jax.ShapeDtypeStruct((2, NPAD, 2), f32),
        mesh=plsc.VectorSubcoreMesh(
            core_axis_name="c", subcore_axis_name="s"),
        compiler_params=pltpu.CompilerParams(use_tc_tiling_on_sc=False),
        scratch_types=[
            pltpu.VMEM_SHARED((NPAD, 2), f32),
            pltpu.VMEM_SHARED((NPAD, 2), f32),
            pltpu.VMEM_SHARED((NPAD, 2), f32),
            pltpu.VMEM((GRP_PER_TILE, GW), jnp.int32),
            pltpu.VMEM((GRP_PER_TILE, GW), jnp.int32),
            pltpu.VMEM((NB, GW, 2), f32),
            pltpu.VMEM((128, 2), f32),
            pltpu.VMEM((IDPER, IDCHUNK), jnp.int32),
            pltpu.SemaphoreType.DMA,
            pltpu.SemaphoreType.DMA,
        ],
    )
    y3 = sc_agg(z3, row_r, col_r, zeros2, ident, bconst)
    y3 = y3[:, :N_NODES, :]

    return pl.pallas_call(
        _softmax_body,
        out_shape=jax.ShapeDtypeStruct((N_NODES, 4), f32),
    )(y3, c2)


# speedup vs baseline: 1.5607x; 1.1107x over previous
import functools

import jax
import jax.numpy as jnp
from jax import lax
from jax.experimental import pallas as pl
from jax.experimental.pallas import tpu as pltpu
from jax.experimental.pallas import tpu_sc as plsc

N_NODES = 10000
N_EDGES = 320000

NC = 2
NS = 16
GW = 80
NGRP = N_EDGES // GW
GRP_PER_TILE = NGRP // NS
NB = 25
NPAD = 10240
ROWS_PER_TILE = NPAD // NS
IDCHUNK = 128
IDPER = ROWS_PER_TILE // IDCHUNK


def _dense_body(x, W0, b0, gw1, W1, b1, gw2, W2, b2, W3, b3, z3, b1p, c2):
    h0 = jnp.dot(x[...], W0[...], preferred_element_type=jnp.float32) + b0[...]
    t1 = jnp.dot(h0 * gw1[...], W1[...], preferred_element_type=jnp.float32)
    t2 = jnp.dot(t1 * gw2[...], W2[...], preferred_element_type=jnp.float32)
    z = jnp.dot(t2, W3[...], preferred_element_type=jnp.float32)
    z3[0, 0:N_NODES, :] = z[:, 0:2]
    z3[1, 0:N_NODES, :] = z[:, 2:4]
    b1v = b1[...].reshape(1, -1)
    bp = jnp.dot(
        jnp.dot(b1v * gw2[...], W2[...], preferred_element_type=jnp.float32),
        W3[...], preferred_element_type=jnp.float32)
    b1p[...] = bp
    c2[...] = jnp.dot(b2[...].reshape(1, -1), W3[...],
                      preferred_element_type=jnp.float32) + b3[...].reshape(1, -1)


def _agg_round(src_sp, dst_sp, colbuf, rowbuf, vals, gsem, ssem):
    def superbatch(i, _):
        gds = []
        for j in range(NB):
            g = i * NB + j
            gds.append(pltpu.async_copy(
                src_sp.at[colbuf.at[g]], vals.at[j], gsem))
        for d in gds:
            d.wait()
        sds = []
        for j in range(NB):
            g = i * NB + j
            sds.append(pltpu.async_copy(
                vals.at[j], dst_sp.at[rowbuf.at[g]], ssem, add=True))
        for d in sds:
            d.wait()
        return 0

    lax.fori_loop(0, GRP_PER_TILE // NB, superbatch, 0)


def _sc_body(z3, rowr, colr, zeros2, ident, bconst, out,
             zsp, y1sp, y2sp, colbuf, rowbuf, vals, bcbuf, idbuf, gsem, ssem):
    c = lax.axis_index("c")
    s = lax.axis_index("s")
    rbase = s * ROWS_PER_TILE

    pltpu.sync_copy(z3.at[c, pl.ds(rbase, ROWS_PER_TILE), :],
                    zsp.at[pl.ds(rbase, ROWS_PER_TILE), :])
    pltpu.sync_copy(zeros2.at[pl.ds(rbase, ROWS_PER_TILE), :],
                    y1sp.at[pl.ds(rbase, ROWS_PER_TILE), :])
    pltpu.sync_copy(zeros2.at[pl.ds(rbase, ROWS_PER_TILE), :],
                    y2sp.at[pl.ds(rbase, ROWS_PER_TILE), :])
    pltpu.sync_copy(colr.at[s], colbuf)
    pltpu.sync_copy(rowr.at[s], rowbuf)
    pltpu.sync_copy(bconst.at[c], bcbuf)
    pltpu.sync_copy(ident.at[s], idbuf)
    plsc.subcore_barrier()

    _agg_round(zsp, y1sp, colbuf, rowbuf, vals, gsem, ssem)
    plsc.subcore_barrier()

    for t in range(IDPER):
        pltpu.async_copy(bcbuf, y1sp.at[idbuf.at[t]], ssem, add=True).wait()
    plsc.subcore_barrier()

    _agg_round(y1sp, y2sp, colbuf, rowbuf, vals, gsem, ssem)
    plsc.subcore_barrier()

    pltpu.sync_copy(y2sp.at[pl.ds(rbase, ROWS_PER_TILE), :],
                    out.at[c, pl.ds(rbase, ROWS_PER_TILE), :])


def _softmax_body(y3, c2, out):
    logits = jnp.concatenate([y3[0, 0:N_NODES, :], y3[1, 0:N_NODES, :]],
                             axis=-1) + c2[...]
    m = jnp.max(logits, axis=-1, keepdims=True)
    e = jnp.exp(logits - m)
    out[...] = e / jnp.sum(e, axis=-1, keepdims=True)


def kernel(x, edge_index, W0, b0, gw1, W1, b1, gw2, W2, b2, W3, b3):
    f32 = jnp.float32

    z3, b1p, c2 = pl.pallas_call(
        _dense_body,
        out_shape=(
            jax.ShapeDtypeStruct((2, NPAD, 2), f32),
            jax.ShapeDtypeStruct((1, 4), f32),
            jax.ShapeDtypeStruct((1, 4), f32),
        ),
    )(x, W0, b0, gw1, W1, b1, gw2, W2, b2, W3, b3)

    row_r = edge_index[0].astype(jnp.int32).reshape(NS, GRP_PER_TILE, GW)
    col_r = edge_index[1].astype(jnp.int32).reshape(NS, GRP_PER_TILE, GW)
    zeros2 = jnp.zeros((NPAD, 2), f32)
    ident = jnp.arange(NPAD, dtype=jnp.int32).reshape(NS, IDPER, IDCHUNK)
    bconst = jnp.broadcast_to(b1p.reshape(2, 1, 2), (2, 128, 2))

    sc_agg = pl.kernel(
        _sc_body,
        out_type=jax.ShapeDtypeStruct((2, NPAD, 2), f32),
        mesh=plsc.VectorSubcoreMesh(
            core_axis_name="c", subcore_axis_name="s"),
        compiler_params=pltpu.CompilerParams(use_tc_tiling_on_sc=False),
        scratch_types=[
            pltpu.VMEM_SHARED((NPAD, 2), f32),
            pltpu.VMEM_SHARED((NPAD, 2), f32),
            pltpu.VMEM_SHARED((NPAD, 2), f32),
            pltpu.VMEM((GRP_PER_TILE, GW), jnp.int32),
            pltpu.VMEM((GRP_PER_TILE, GW), jnp.int32),
            pltpu.VMEM((NB, GW, 2), f32),
            pltpu.VMEM((128, 2), f32),
            pltpu.VMEM((IDPER, IDCHUNK), jnp.int32),
            pltpu.SemaphoreType.DMA,
            pltpu.SemaphoreType.DMA,
        ],
    )
    y3 = sc_agg(z3, row_r, col_r, zeros2, ident, bconst)

    return pl.pallas_call(
        _softmax_body,
        out_shape=jax.ShapeDtypeStruct((N_NODES, 4), f32),
    )(y3, c2)


# speedup vs baseline: 1.6517x; 1.0583x over previous
import functools

import jax
import jax.numpy as jnp
from jax import lax
from jax.experimental import pallas as pl
from jax.experimental.pallas import tpu as pltpu
from jax.experimental.pallas import tpu_sc as plsc

N_NODES = 10000
N_EDGES = 320000

NC = 2
NS = 16
GW = 80
NGRP = N_EDGES // GW
GRP_PER_TILE = NGRP // NS
NB = 50
NPAD = 10240
ROWS_PER_TILE = NPAD // NS
IDCHUNK = 128
IDPER = ROWS_PER_TILE // IDCHUNK


def _dense_body(x, W0, b0, gw1, W1, b1, gw2, W2, b2, W3, b3, z3, b1p, c2):
    h0 = jnp.dot(x[...], W0[...], preferred_element_type=jnp.float32) + b0[...]
    t1 = jnp.dot(h0 * gw1[...], W1[...], preferred_element_type=jnp.float32)
    t2 = jnp.dot(t1 * gw2[...], W2[...], preferred_element_type=jnp.float32)
    z = jnp.dot(t2, W3[...], preferred_element_type=jnp.float32)
    z3[0, 0:N_NODES, :] = z[:, 0:2]
    z3[1, 0:N_NODES, :] = z[:, 2:4]
    b1v = b1[...].reshape(1, -1)
    bp = jnp.dot(
        jnp.dot(b1v * gw2[...], W2[...], preferred_element_type=jnp.float32),
        W3[...], preferred_element_type=jnp.float32)
    b1p[...] = bp
    c2[...] = jnp.dot(b2[...].reshape(1, -1), W3[...],
                      preferred_element_type=jnp.float32) + b3[...].reshape(1, -1)


def _agg_round(src_sp, dst_sp, colbuf, rowbuf, vals, gsem, ssem):
    def superbatch(i, _):
        gds = []
        for j in range(NB):
            g = i * NB + j
            gds.append(pltpu.async_copy(
                src_sp.at[colbuf.at[g]], vals.at[j], gsem))
        for d in gds:
            d.wait()
        sds = []
        for j in range(NB):
            g = i * NB + j
            sds.append(pltpu.async_copy(
                vals.at[j], dst_sp.at[rowbuf.at[g]], ssem, add=True))
        for d in sds:
            d.wait()
        return 0

    lax.fori_loop(0, GRP_PER_TILE // NB, superbatch, 0)


def _sc_body(z3, rowr, colr, zeros2, ident, bconst, out,
             zsp, y1sp, y2sp, colbuf, rowbuf, vals, bcbuf, idbuf, gsem, ssem):
    c = lax.axis_index("c")
    s = lax.axis_index("s")
    rbase = s * ROWS_PER_TILE

    pltpu.sync_copy(z3.at[c, pl.ds(rbase, ROWS_PER_TILE), :],
                    zsp.at[pl.ds(rbase, ROWS_PER_TILE), :])
    pltpu.sync_copy(zeros2.at[pl.ds(rbase, ROWS_PER_TILE), :],
                    y1sp.at[pl.ds(rbase, ROWS_PER_TILE), :])
    pltpu.sync_copy(zeros2.at[pl.ds(rbase, ROWS_PER_TILE), :],
                    y2sp.at[pl.ds(rbase, ROWS_PER_TILE), :])
    pltpu.sync_copy(colr.at[s], colbuf)
    pltpu.sync_copy(rowr.at[s], rowbuf)
    pltpu.sync_copy(bconst.at[c], bcbuf)
    pltpu.sync_copy(ident.at[s], idbuf)
    plsc.subcore_barrier()

    _agg_round(zsp, y1sp, colbuf, rowbuf, vals, gsem, ssem)
    plsc.subcore_barrier()

    for t in range(IDPER):
        pltpu.async_copy(bcbuf, y1sp.at[idbuf.at[t]], ssem, add=True).wait()
    plsc.subcore_barrier()

    _agg_round(y1sp, y2sp, colbuf, rowbuf, vals, gsem, ssem)
    plsc.subcore_barrier()

    pltpu.sync_copy(y2sp.at[pl.ds(rbase, ROWS_PER_TILE), :],
                    out.at[c, pl.ds(rbase, ROWS_PER_TILE), :])


def _softmax_body(y3, c2, out):
    logits = jnp.concatenate([y3[0, 0:N_NODES, :], y3[1, 0:N_NODES, :]],
                             axis=-1) + c2[...]
    m = jnp.max(logits, axis=-1, keepdims=True)
    e = jnp.exp(logits - m)
    out[...] = e / jnp.sum(e, axis=-1, keepdims=True)


def kernel(x, edge_index, W0, b0, gw1, W1, b1, gw2, W2, b2, W3, b3):
    f32 = jnp.float32

    z3, b1p, c2 = pl.pallas_call(
        _dense_body,
        out_shape=(
            jax.ShapeDtypeStruct((2, NPAD, 2), f32),
            jax.ShapeDtypeStruct((1, 4), f32),
            jax.ShapeDtypeStruct((1, 4), f32),
        ),
    )(x, W0, b0, gw1, W1, b1, gw2, W2, b2, W3, b3)

    row_r = edge_index[0].astype(jnp.int32).reshape(NS, GRP_PER_TILE, GW)
    col_r = edge_index[1].astype(jnp.int32).reshape(NS, GRP_PER_TILE, GW)
    zeros2 = jnp.zeros((NPAD, 2), f32)
    ident = jnp.arange(NPAD, dtype=jnp.int32).reshape(NS, IDPER, IDCHUNK)
    bconst = jnp.broadcast_to(b1p.reshape(2, 1, 2), (2, 128, 2))

    sc_agg = pl.kernel(
        _sc_body,
        out_type=jax.ShapeDtypeStruct((2, NPAD, 2), f32),
        mesh=plsc.VectorSubcoreMesh(
            core_axis_name="c", subcore_axis_name="s"),
        compiler_params=pltpu.CompilerParams(use_tc_tiling_on_sc=False),
        scratch_types=[
            pltpu.VMEM_SHARED((NPAD, 2), f32),
            pltpu.VMEM_SHARED((NPAD, 2), f32),
            pltpu.VMEM_SHARED((NPAD, 2), f32),
            pltpu.VMEM((GRP_PER_TILE, GW), jnp.int32),
            pltpu.VMEM((GRP_PER_TILE, GW), jnp.int32),
            pltpu.VMEM((NB, GW, 2), f32),
            pltpu.VMEM((128, 2), f32),
            pltpu.VMEM((IDPER, IDCHUNK), jnp.int32),
            pltpu.SemaphoreType.DMA,
            pltpu.SemaphoreType.DMA,
        ],
    )
    y3 = sc_agg(z3, row_r, col_r, zeros2, ident, bconst)

    return pl.pallas_call(
        _softmax_body,
        out_shape=jax.ShapeDtypeStruct((N_NODES, 4), f32),
    )(y3, c2)


# speedup vs baseline: 1.6773x; 1.0155x over previous
import functools

import jax
import jax.numpy as jnp
from jax import lax
from jax.experimental import pallas as pl
from jax.experimental.pallas import tpu as pltpu
from jax.experimental.pallas import tpu_sc as plsc

N_NODES = 10000
N_EDGES = 320000

NC = 2
NS = 16
GW = 80
NGRP = N_EDGES // GW
GRP_PER_TILE = NGRP // NS
NB = 125
NPAD = 10240
ROWS_PER_TILE = NPAD // NS
IDCHUNK = 128
IDPER = ROWS_PER_TILE // IDCHUNK


def _dense_body(x, W0, b0, gw1, W1, b1, gw2, W2, b2, W3, b3, z3, b1p, c2):
    h0 = jnp.dot(x[...], W0[...], preferred_element_type=jnp.float32) + b0[...]
    t1 = jnp.dot(h0 * gw1[...], W1[...], preferred_element_type=jnp.float32)
    t2 = jnp.dot(t1 * gw2[...], W2[...], preferred_element_type=jnp.float32)
    z = jnp.dot(t2, W3[...], preferred_element_type=jnp.float32)
    z3[0, 0:N_NODES, :] = z[:, 0:2]
    z3[1, 0:N_NODES, :] = z[:, 2:4]
    b1v = b1[...].reshape(1, -1)
    bp = jnp.dot(
        jnp.dot(b1v * gw2[...], W2[...], preferred_element_type=jnp.float32),
        W3[...], preferred_element_type=jnp.float32)
    b1p[...] = bp
    c2[...] = jnp.dot(b2[...].reshape(1, -1), W3[...],
                      preferred_element_type=jnp.float32) + b3[...].reshape(1, -1)


def _agg_round(src_sp, dst_sp, colbuf, rowbuf, vals, gsem, ssem):
    def superbatch(i, _):
        gds = []
        for j in range(NB):
            g = i * NB + j
            gds.append(pltpu.async_copy(
                src_sp.at[colbuf.at[g]], vals.at[j], gsem))
        for d in gds:
            d.wait()
        sds = []
        for j in range(NB):
            g = i * NB + j
            sds.append(pltpu.async_copy(
                vals.at[j], dst_sp.at[rowbuf.at[g]], ssem, add=True))
        for d in sds:
            d.wait()
        return 0

    lax.fori_loop(0, GRP_PER_TILE // NB, superbatch, 0)


def _sc_body(z3, rowr, colr, zeros2, ident, bconst, out,
             zsp, y1sp, y2sp, colbuf, rowbuf, vals, bcbuf, idbuf, gsem, ssem):
    c = lax.axis_index("c")
    s = lax.axis_index("s")
    rbase = s * ROWS_PER_TILE

    pltpu.sync_copy(z3.at[c, pl.ds(rbase, ROWS_PER_TILE), :],
                    zsp.at[pl.ds(rbase, ROWS_PER_TILE), :])
    pltpu.sync_copy(zeros2.at[pl.ds(rbase, ROWS_PER_TILE), :],
                    y1sp.at[pl.ds(rbase, ROWS_PER_TILE), :])
    pltpu.sync_copy(zeros2.at[pl.ds(rbase, ROWS_PER_TILE), :],
                    y2sp.at[pl.ds(rbase, ROWS_PER_TILE), :])
    pltpu.sync_copy(colr.at[s], colbuf)
    pltpu.sync_copy(rowr.at[s], rowbuf)
    pltpu.sync_copy(bconst.at[c], bcbuf)
    pltpu.sync_copy(ident.at[s], idbuf)
    plsc.subcore_barrier()

    _agg_round(zsp, y1sp, colbuf, rowbuf, vals, gsem, ssem)
    plsc.subcore_barrier()

    for t in range(IDPER):
        pltpu.async_copy(bcbuf, y1sp.at[idbuf.at[t]], ssem, add=True).wait()
    plsc.subcore_barrier()

    _agg_round(y1sp, y2sp, colbuf, rowbuf, vals, gsem, ssem)
    plsc.subcore_barrier()

    pltpu.sync_copy(y2sp.at[pl.ds(rbase, ROWS_PER_TILE), :],
                    out.at[c, pl.ds(rbase, ROWS_PER_TILE), :])


def _softmax_body(y3, c2, out):
    logits = jnp.concatenate([y3[0, 0:N_NODES, :], y3[1, 0:N_NODES, :]],
                             axis=-1) + c2[...]
    m = jnp.max(logits, axis=-1, keepdims=True)
    e = jnp.exp(logits - m)
    out[...] = e / jnp.sum(e, axis=-1, keepdims=True)


def kernel(x, edge_index, W0, b0, gw1, W1, b1, gw2, W2, b2, W3, b3):
    f32 = jnp.float32

    z3, b1p, c2 = pl.pallas_call(
        _dense_body,
        out_shape=(
            jax.ShapeDtypeStruct((2, NPAD, 2), f32),
            jax.ShapeDtypeStruct((1, 4), f32),
            jax.ShapeDtypeStruct((1, 4), f32),
        ),
    )(x, W0, b0, gw1, W1, b1, gw2, W2, b2, W3, b3)

    row_r = edge_index[0].astype(jnp.int32).reshape(NS, GRP_PER_TILE, GW)
    col_r = edge_index[1].astype(jnp.int32).reshape(NS, GRP_PER_TILE, GW)
    zeros2 = jnp.zeros((NPAD, 2), f32)
    ident = jnp.arange(NPAD, dtype=jnp.int32).reshape(NS, IDPER, IDCHUNK)
    bconst = jnp.broadcast_to(b1p.reshape(2, 1, 2), (2, 128, 2))

    sc_agg = pl.kernel(
        _sc_body,
        out_type=jax.ShapeDtypeStruct((2, NPAD, 2), f32),
        mesh=plsc.VectorSubcoreMesh(
            core_axis_name="c", subcore_axis_name="s"),
        compiler_params=pltpu.CompilerParams(use_tc_tiling_on_sc=False),
        scratch_types=[
            pltpu.VMEM_SHARED((NPAD, 2), f32),
            pltpu.VMEM_SHARED((NPAD, 2), f32),
            pltpu.VMEM_SHARED((NPAD, 2), f32),
            pltpu.VMEM((GRP_PER_TILE, GW), jnp.int32),
            pltpu.VMEM((GRP_PER_TILE, GW), jnp.int32),
            pltpu.VMEM((NB, GW, 2), f32),
            pltpu.VMEM((128, 2), f32),
            pltpu.VMEM((IDPER, IDCHUNK), jnp.int32),
            pltpu.SemaphoreType.DMA,
            pltpu.SemaphoreType.DMA,
        ],
    )
    y3 = sc_agg(z3, row_r, col_r, zeros2, ident, bconst)

    return pl.pallas_call(
        _softmax_body,
        out_shape=jax.ShapeDtypeStruct((N_NODES, 4), f32),
    )(y3, c2)
